# baseline (speedup 1.0000x reference)
import functools

import jax
import jax.numpy as jnp
from jax import lax
from jax.experimental import pallas as pl
from jax.experimental.pallas import tpu as pltpu
from jax.experimental.pallas import tpu_sc as plsc

D_MODEL = 512
NUM_TILES = 16
TOP_K = 4
B = 4096
TERNARY_W = 0.01
SPARSITY_W = 0.005

TBLK_A = 512
TBLK_B = 512
RBLK = 512
NASSIGN = B * TOP_K
NB = (NASSIGN + NUM_TILES * (RBLK - 1)) // RBLK + 1
NPAD = NB * RBLK
NEG_INF = -3.0e38


def _dot(a, b):
    return jax.lax.dot_general(a, b, (((1,), (0,)), ((), ())),
                               preferred_element_type=jnp.float32)


def _prep_body(ints_ref, opP_ref, sa_ref, sb_ref, sc_ref, w_in_ref,
               b_in_ref, w_r_ref, b_r_ref, tri_ref, eyeT_ref,
               x_ref, topi_ref, topn_ref, lr_ref, topiT_ref, lrT_ref,
               il_ref, run_ref):
    i = pl.program_id(0)
    ints = ints_ref[...]
    op = ints[:, 0:1]
    a = ints[:, 1:2]
    b = ints[:, 2:3]
    c = ints[:, 3:4]
    T = ints.shape[0]

    @pl.when(i == 0)
    def _():
        il_ref[...] = jnp.zeros_like(il_ref)
        run_ref[...] = jnp.zeros_like(run_ref)

    ones16 = jnp.ones((NUM_TILES, 1), jnp.float32)

    oh8 = (op == lax.broadcasted_iota(jnp.int32, (1, 8), 1)
           ).astype(jnp.float32)
    bit_iota = lax.broadcasted_iota(jnp.int32, (1, 8), 1)
    a_bits = ((lax.shift_right_logical(a, bit_iota)) & 1).astype(jnp.float32)
    b_bits = ((lax.shift_right_logical(b, bit_iota)) & 1).astype(jnp.float32)
    feats = (_dot(oh8, opP_ref[...]) + _dot(a_bits, sa_ref[...]) +
             _dot(b_bits, sb_ref[...]) +
             _dot(c.astype(jnp.float32), sc_ref[...]))
    x = _dot(feats, w_in_ref[...]) + b_in_ref[...]
    x_ref[...] = x

    logits = _dot(x, w_r_ref[...]) + b_r_ref[...]
    m = jnp.max(logits, axis=1, keepdims=True)
    e = jnp.exp(logits - m)
    gates = e / _dot(e, ones16)

    iota16 = lax.broadcasted_iota(jnp.int32, (T, NUM_TILES), 1)
    v = gates
    tis, tvs = [], []
    for _ in range(TOP_K):
        mx = jnp.max(v, axis=1, keepdims=True)
        idx = jnp.min(jnp.where(v == mx, iota16, NUM_TILES), axis=1,
                      keepdims=True)
        tis.append(idx)
        tvs.append(mx)
        v = jnp.where(iota16 == idx, NEG_INF, v)
    topi = jnp.concatenate(tis, axis=1)
    topv = jnp.concatenate(tvs, axis=1)
    topn = topv / _dot(topv, jnp.ones((TOP_K, 1), jnp.float32))
    topi_ref[...] = topi
    topn_ref[...] = topn

    ohk = [(iota16 == topi[:, k:k + 1]).astype(jnp.float32)
           for k in range(TOP_K)]
    disp = ohk[0] + ohk[1] + ohk[2] + ohk[3]

    ct = _dot(tri_ref[...], disp)
    base = ct + run_ref[0:1, 0:16]
    prev = jnp.zeros((T, NUM_TILES), jnp.float32)
    lrs = []
    for k in range(TOP_K):
        lrs.append(_dot(ohk[k] * (base + prev), ones16))
        prev = prev + ohk[k]
    lr_f = jnp.concatenate(lrs, axis=1)
    lr_ref[...] = lr_f.astype(jnp.int32)
    run_ref[0:1, 0:16] += jnp.sum(disp, axis=0, keepdims=True)

    topiT_ref[...] = jax.lax.dot_general(
        topi.astype(jnp.float32), eyeT_ref[...],
        (((0,), (0,)), ((), ())),
        preferred_element_type=jnp.float32).astype(jnp.int32)
    lrT_ref[...] = jax.lax.dot_general(
        lr_f, eyeT_ref[...], (((0,), (0,)), ((), ())),
        preferred_element_type=jnp.float32).astype(jnp.int32)

    ones_col = jnp.ones((T, 1), jnp.float32)
    impcol = jax.lax.dot_general(gates, ones_col, (((0,), (0,)), ((), ())),
                                 preferred_element_type=jnp.float32)
    cntcol = jax.lax.dot_general(disp, ones_col, (((0,), (0,)), ((), ())),
                                 preferred_element_type=jnp.float32)
    il_ref[0:NUM_TILES, 0:1] += impcol
    il_ref[0:NUM_TILES, 1:2] += cntcol


def _prep_call(ints, opP, sa, sb, sc, w_in_p, b_in, w_r, b_r, tri, eyeT):
    nblk = B // TBLK_A

    def cst(s):
        return pl.BlockSpec(s, lambda i: tuple(0 for _ in s))

    def blk(s):
        return pl.BlockSpec(s, lambda i: (i,) + tuple(0 for _ in s[1:]))

    return pl.pallas_call(
        _prep_body,
        grid=(nblk,),
        in_specs=[
            blk((TBLK_A, 4)),
            cst((8, 128)),
            cst((8, 128)),
            cst((8, 128)),
            cst((1, 128)),
            cst((128, D_MODEL)),
            cst((1, D_MODEL)),
            cst((D_MODEL, NUM_TILES)),
            cst((1, NUM_TILES)),
            cst((TBLK_A, TBLK_A)),
            cst((TBLK_A, TBLK_A)),
        ],
        out_specs=[
            blk((TBLK_A, D_MODEL)),
            blk((TBLK_A, TOP_K)),
            blk((TBLK_A, TOP_K)),
            blk((TBLK_A, TOP_K)),
            pl.BlockSpec((TOP_K, TBLK_A), lambda i: (0, i)),
            pl.BlockSpec((TOP_K, TBLK_A), lambda i: (0, i)),
            cst((NUM_TILES, 128)),
        ],
        out_shape=[
            jax.ShapeDtypeStruct((B, D_MODEL), jnp.float32),
            jax.ShapeDtypeStruct((B, TOP_K), jnp.int32),
            jax.ShapeDtypeStruct((B, TOP_K), jnp.float32),
            jax.ShapeDtypeStruct((B, TOP_K), jnp.int32),
            jax.ShapeDtypeStruct((TOP_K, B), jnp.int32),
            jax.ShapeDtypeStruct((TOP_K, B), jnp.int32),
            jax.ShapeDtypeStruct((NUM_TILES, 128), jnp.float32),
        ],
        scratch_shapes=[pltpu.VMEM((8, 128), jnp.float32)],
    )(ints, opP, sa, sb, sc, w_in_p, b_in, w_r, b_r, tri, eyeT)


def _plan_body(topi_ref, lr_ref, topiT_ref, lrT_ref, il_ref,
               p_ref, pT_ref, plan_ref):
    cnt = il_ref[0:NUM_TILES, 1:2].astype(jnp.int32)
    cnt_pad = ((cnt + (RBLK - 1)) // RBLK) * RBLK
    tri16 = (lax.broadcasted_iota(jnp.int32, (NUM_TILES, NUM_TILES), 0) >
             lax.broadcasted_iota(jnp.int32, (NUM_TILES, NUM_TILES), 1)
             ).astype(jnp.float32)
    off_col = jax.lax.dot_general(
        tri16, cnt_pad.astype(jnp.float32), (((1,), (0,)), ((), ())),
        preferred_element_type=jnp.float32).astype(jnp.int32)

    eye16 = (lax.broadcasted_iota(jnp.int32, (NUM_TILES, NUM_TILES), 0) ==
             lax.broadcasted_iota(jnp.int32, (NUM_TILES, NUM_TILES), 1)
             ).astype(jnp.float32)
    off_row = jax.lax.dot_general(
        off_col.astype(jnp.float32), eye16, (((0,), (0,)), ((), ())),
        preferred_element_type=jnp.float32)
    thr = (lax.broadcasted_iota(jnp.int32, (NB, 1), 0) * RBLK
           ).astype(jnp.float32)
    cmp = (off_row <= thr).astype(jnp.float32)
    be_col = jax.lax.dot_general(
        cmp, jnp.ones((NUM_TILES, 1), jnp.float32),
        (((1,), (0,)), ((), ())),
        preferred_element_type=jnp.float32).astype(jnp.int32) - 1
    plan_ref[...] = jnp.zeros_like(plan_ref)
    plan_ref[:, 0:1] = be_col
    na = jnp.sum(cnt_pad) // RBLK
    plan_ref[0:1, 1:2] = jnp.reshape(na, (1, 1))
    plan_ref[0:NUM_TILES, 2:3] = off_col

    topi = topi_ref[...]
    offsel = jnp.zeros(topi.shape, jnp.int32)
    for e in range(NUM_TILES):
        offsel = offsel + jnp.where(topi == e, off_col[e, 0], 0)
    p_ref[...] = offsel + lr_ref[...]

    topiT = topiT_ref[...]
    offselT = jnp.zeros(topiT.shape, jnp.int32)
    for e in range(NUM_TILES):
        offselT = offselT + jnp.where(topiT == e, off_col[e, 0], 0)
    pT_ref[...] = offselT + lrT_ref[...]


def _plan_call(topi, lr, topiT, lrT, il):
    return pl.pallas_call(
        _plan_body,
        grid=(1,),
        in_specs=[
            pl.BlockSpec((B, TOP_K), lambda i: (0, 0)),
            pl.BlockSpec((B, TOP_K), lambda i: (0, 0)),
            pl.BlockSpec((TOP_K, B), lambda i: (0, 0)),
            pl.BlockSpec((TOP_K, B), lambda i: (0, 0)),
            pl.BlockSpec((NUM_TILES, 128), lambda i: (0, 0)),
        ],
        out_specs=[
            pl.BlockSpec((B, TOP_K), lambda i: (0, 0)),
            pl.BlockSpec((TOP_K, B), lambda i: (0, 0)),
            pl.BlockSpec((NB, 128), lambda i: (0, 0)),
        ],
        out_shape=[
            jax.ShapeDtypeStruct((B, TOP_K), jnp.int32),
            jax.ShapeDtypeStruct((TOP_K, B), jnp.int32),
            jax.ShapeDtypeStruct((NB, 128), jnp.int32),
        ],
    )(topi, lr, topiT, lrT, il)


SC_TOK = 64


def _make_scatter():
    info = plsc.get_sparse_core_info()
    nw = info.num_cores * info.num_subcores
    tok_per_w = B // nw
    nchunk = tok_per_w // SC_TOK
    mesh = plsc.VectorSubcoreMesh(core_axis_name="c", subcore_axis_name="s")

    @functools.partial(
        pl.kernel, mesh=mesh,
        out_type=jax.ShapeDtypeStruct((NPAD, D_MODEL), jnp.float32),
        scratch_types=[
            pltpu.VMEM((2, TOP_K, SC_TOK), jnp.int32),
            pltpu.VMEM((2, SC_TOK, D_MODEL), jnp.float32),
            pltpu.SemaphoreType.DMA,
            pltpu.SemaphoreType.DMA,
            pltpu.SemaphoreType.DMA,
        ],
    )
    def scatter_k(x_hbm, pT_hbm, xs_hbm, pidx_v, rows_v, sem_l, sem_i,
                  sem_s):
        wid = lax.axis_index("s") * info.num_cores + lax.axis_index("c")

        def load(ci, bb):
            tb = wid * tok_per_w + ci * SC_TOK
            dl = pltpu.async_copy(x_hbm.at[pl.ds(tb, SC_TOK)],
                                  rows_v.at[bb], sem_l)
            dis = [pltpu.async_copy(pT_hbm.at[k, pl.ds(tb, SC_TOK)],
                                    pidx_v.at[bb, k], sem_i)
                   for k in range(TOP_K)]
            return [dl] + dis

        pend = load(0, 0)
        scat = []
        for ci in range(nchunk):
            bb = ci & 1
            for d in pend:
                d.wait()
            if ci + 1 < nchunk:
                pend = load(ci + 1, 1 - bb)
            for d in scat:
                d.wait()
            scat = [pltpu.async_copy(rows_v.at[bb],
                                     xs_hbm.at[pidx_v.at[bb, k]], sem_s)
                    for k in range(TOP_K)]
        for d in scat:
            d.wait()

    return scatter_k


def _ffn_body(be_ref, na_ref, xs_ref, w1_ref, b1_ref, w2_ref, b2_ref,
              ys_ref):
    i = pl.program_id(0)

    @pl.when(i < na_ref[0])
    def _():
        h = jnp.maximum(
            _dot(xs_ref[...].astype(jnp.bfloat16), w1_ref[0]) + b1_ref[0],
            0.0)
        ys_ref[...] = _dot(h.astype(jnp.bfloat16), w2_ref[0]) + b2_ref[0]


def _ffn_call(be, na, xs, w1, b1, w2, b2):
    grid_spec = pltpu.PrefetchScalarGridSpec(
        num_scalar_prefetch=2,
        grid=(NB,),
        in_specs=[
            pl.BlockSpec((RBLK, D_MODEL), lambda i, be, na: (i, 0)),
            pl.BlockSpec((1, D_MODEL, D_MODEL),
                         lambda i, be, na: (be[i], 0, 0)),
            pl.BlockSpec((1, 1, D_MODEL), lambda i, be, na: (be[i], 0, 0)),
            pl.BlockSpec((1, D_MODEL, D_MODEL),
                         lambda i, be, na: (be[i], 0, 0)),
            pl.BlockSpec((1, 1, D_MODEL), lambda i, be, na: (be[i], 0, 0)),
        ],
        out_specs=pl.BlockSpec((RBLK, D_MODEL), lambda i, be, na: (i, 0)),
    )
    return pl.pallas_call(
        _ffn_body,
        grid_spec=grid_spec,
        out_shape=jax.ShapeDtypeStruct((NPAD, D_MODEL), jnp.float32),
    )(be, na, xs, w1, b1, w2, b2)


GA_CHUNK = 64


def _make_gather():
    info = plsc.get_sparse_core_info()
    nw = info.num_cores * info.num_subcores
    rows_per_w = NASSIGN // nw
    nchunk = rows_per_w // GA_CHUNK
    mesh = plsc.VectorSubcoreMesh(core_axis_name="c", subcore_axis_name="s")

    @functools.partial(
        pl.kernel, mesh=mesh,
        out_type=jax.ShapeDtypeStruct((NASSIGN, D_MODEL), jnp.float32),
        scratch_types=[
            pltpu.VMEM((2, GA_CHUNK), jnp.int32),
            pltpu.VMEM((2, GA_CHUNK, D_MODEL), jnp.float32),
            pltpu.SemaphoreType.DMA,
            pltpu.SemaphoreType.DMA,
            pltpu.SemaphoreType.DMA,
        ],
    )
    def gather_k(ys_hbm, p_hbm, y4_hbm, pidx_v, rows_v, sem_i, sem_g,
                 sem_w):
        wid = lax.axis_index("s") * info.num_cores + lax.axis_index("c")
        base = wid * rows_per_w

        def idx_load(ci, bb):
            return pltpu.async_copy(
                p_hbm.at[pl.ds(base + ci * GA_CHUNK, GA_CHUNK)],
                pidx_v.at[bb], sem_i)

        def gather_start(bb):
            return pltpu.async_copy(ys_hbm.at[pidx_v.at[bb]],
                                    rows_v.at[bb], sem_g)

        def write_start(ci, bb):
            return pltpu.async_copy(
                rows_v.at[bb],
                y4_hbm.at[pl.ds(base + ci * GA_CHUNK, GA_CHUNK)], sem_w)

        idx_load(0, 0).wait()
        gat = gather_start(0)
        idx_pend = idx_load(1, 1)
        wr = None
        for ci in range(nchunk):
            bb = ci & 1
            gat.wait()
            if ci + 1 < nchunk:
                idx_pend.wait()
                if wr is not None:
                    wr.wait()
                gat = gather_start(1 - bb)
                if ci + 2 < nchunk:
                    idx_pend = idx_load(ci + 2, bb)
            elif wr is not None:
                wr.wait()
            wr = write_start(ci, bb)
        wr.wait()

    return gather_k


def _aux_body(w1_ref, w2_ref, il_ref, out_ref):
    e = pl.program_id(0)

    @pl.when(e == 0)
    def _():
        out_ref[...] = jnp.zeros_like(out_ref)

    aw1 = jnp.abs(w1_ref[0])
    aw2 = jnp.abs(w2_ref[0])
    s = (jnp.sum(aw1 * jnp.abs(1.0 - aw1)) + jnp.sum(aw2 * jnp.abs(1.0 - aw2)))
    out_ref[0:1, 0:1] += jnp.reshape(s, (1, 1))

    @pl.when(e == NUM_TILES - 1)
    def _():
        imp = il_ref[0:NUM_TILES, 0:1] * (1.0 / B)
        load = il_ref[0:NUM_TILES, 1:2] * (1.0 / B)
        lb = NUM_TILES * jnp.sum(imp * load)
        tern = out_ref[0, 0] / (NUM_TILES * D_MODEL * D_MODEL)
        out_ref[0:1, 0:1] = jnp.reshape(
            SPARSITY_W * lb + TERNARY_W * tern, (1, 1))


def _aux_call(w1, w2, il):
    return pl.pallas_call(
        _aux_body,
        grid=(NUM_TILES,),
        in_specs=[
            pl.BlockSpec((1, D_MODEL, D_MODEL), lambda e: (e, 0, 0)),
            pl.BlockSpec((1, D_MODEL, D_MODEL), lambda e: (e, 0, 0)),
            pl.BlockSpec((NUM_TILES, 128), lambda e: (0, 0)),
        ],
        out_specs=pl.BlockSpec((8, 128), lambda e: (0, 0)),
        out_shape=jax.ShapeDtypeStruct((8, 128), jnp.float32),
    )(w1, w2, il)


def _head_body(y4_ref, topn_ref, wh1_ref, bh1_ref, wh2_ref, bh2_ref,
               w1_ref, w2_ref, il_ref, rb_ref, aux_ref):
    i = pl.program_id(0)

    @pl.when(i < B // TBLK_B)
    def _():
        T = topn_ref.shape[0]
        y3 = y4_ref[...].reshape(T, TOP_K, D_MODEL)
        topn = topn_ref[...]
        out = y3[:, 0, :] * topn[:, 0:1]
        for k in range(1, TOP_K):
            out = out + y3[:, k, :] * topn[:, k:k + 1]
        h = jnp.maximum(_dot(out, wh1_ref[...]) + bh1_ref[...], 0.0)
        z = _dot(h, wh2_ref[...]) + bh2_ref[...]
        rb_ref[...] = 1.0 / (1.0 + jnp.exp(-z))

    @pl.when(i == 0)
    def _():
        aux_ref[...] = jnp.zeros_like(aux_ref)

    aw1 = jnp.abs(w1_ref[0].astype(jnp.float32))
    aw2 = jnp.abs(w2_ref[0].astype(jnp.float32))
    s = (jnp.sum(aw1 * jnp.abs(1.0 - aw1)) + jnp.sum(aw2 * jnp.abs(1.0 - aw2)))
    aux_ref[0:1, 0:1] += jnp.reshape(s, (1, 1))

    @pl.when(i == NUM_TILES - 1)
    def _():
        imp = il_ref[0:NUM_TILES, 0:1] * (1.0 / B)
        load = il_ref[0:NUM_TILES, 1:2] * (1.0 / B)
        lb = NUM_TILES * jnp.sum(imp * load)
        tern = aux_ref[0, 0] / (NUM_TILES * D_MODEL * D_MODEL)
        aux_ref[0:1, 0:1] = jnp.reshape(
            SPARSITY_W * lb + TERNARY_W * tern, (1, 1))


def _head_call(y4, topn, wh1_p, bh1_p, wh2_p, bh2, w1_bf, w2_bf, il):
    nh = B // TBLK_B
    return pl.pallas_call(
        _head_body,
        grid=(NUM_TILES,),
        in_specs=[
            pl.BlockSpec((TBLK_B * TOP_K, D_MODEL),
                         lambda i: (jnp.minimum(i, nh - 1), 0)),
            pl.BlockSpec((TBLK_B, TOP_K),
                         lambda i: (jnp.minimum(i, nh - 1), 0)),
            pl.BlockSpec((D_MODEL, 128), lambda i: (0, 0)),
            pl.BlockSpec((1, 128), lambda i: (0, 0)),
            pl.BlockSpec((128, 8), lambda i: (0, 0)),
            pl.BlockSpec((1, 8), lambda i: (0, 0)),
            pl.BlockSpec((1, D_MODEL, D_MODEL), lambda i: (i, 0, 0)),
            pl.BlockSpec((1, D_MODEL, D_MODEL), lambda i: (i, 0, 0)),
            pl.BlockSpec((NUM_TILES, 128), lambda i: (0, 0)),
        ],
        out_specs=[
            pl.BlockSpec((TBLK_B, 8), lambda i: (jnp.minimum(i, nh - 1), 0)),
            pl.BlockSpec((8, 128), lambda i: (0, 0)),
        ],
        out_shape=[
            jax.ShapeDtypeStruct((B, 8), jnp.float32),
            jax.ShapeDtypeStruct((8, 128), jnp.float32),
        ],
    )(y4, topn, wh1_p, bh1_p, wh2_p, bh2, w1_bf, w2_bf, il)


def kernel(op_idx, a, b, c, op_embed, W_in, b_in, W_router, b_router,
           W1, b1, W2, b2, W_h1, b_h1, W_h2, b_h2):
    ints = jnp.stack([op_idx.astype(jnp.int32), a.astype(jnp.int32),
                      b.astype(jnp.int32), c.astype(jnp.int32)], axis=1)
    w_in_p = jnp.pad(W_in, ((0, 128 - 33), (0, 0)))
    opP = jnp.pad(op_embed, ((0, 0), (0, 128 - 16)))
    ar8 = jnp.arange(8)
    ar128 = jnp.arange(128)
    sa = (ar128[None, :] == (16 + ar8)[:, None]).astype(jnp.float32)
    sb = (ar128[None, :] == (24 + ar8)[:, None]).astype(jnp.float32)
    sc = (ar128[None, :] == 32).astype(jnp.float32).reshape(1, 128)
    wh1_p = jnp.pad(W_h1, ((0, 0), (0, 128 - 32)))
    bh1_p = jnp.pad(b_h1, (0, 128 - 32)).reshape(1, 128)
    wh2_p = jnp.pad(W_h2, ((0, 128 - 32), (0, 0)))
    tri = (jnp.arange(TBLK_A)[:, None] > jnp.arange(TBLK_A)[None, :]
           ).astype(jnp.float32)
    eyeT = jnp.eye(TBLK_A, dtype=jnp.float32)

    x_bf, topi, topn, lr, topiT, lrT, il = _prep_call(
        ints, opP, sa, sb, sc, w_in_p,
        b_in.reshape(1, D_MODEL), W_router, b_router.reshape(1, NUM_TILES),
        tri, eyeT)
    p, pT, plan = _plan_call(topi, lr, topiT, lrT, il)

    p_flat = p.reshape(NASSIGN)
    xs = _make_scatter()(x_bf, pT)
    be = plan[:, 0]
    na = plan[0:1, 1].reshape(1)
    w1_bf = W1.astype(jnp.bfloat16)
    w2_bf = W2.astype(jnp.bfloat16)
    ys = _ffn_call(be, na, xs, w1_bf,
                   b1.reshape(NUM_TILES, 1, D_MODEL),
                   w2_bf,
                   b2.reshape(NUM_TILES, 1, D_MODEL))
    y4 = _make_gather()(ys, p_flat)

    result_bits, auxm = _head_call(y4, topn, wh1_p, bh1_p, wh2_p,
                                   b_h2.reshape(1, 8), w1_bf, w2_bf, il)
    aux = auxm[0, 0]
    return result_bits, topi, aux


# speedup vs baseline: 1.0476x; 1.0476x over previous
import functools

import jax
import jax.numpy as jnp
from jax import lax
from jax.experimental import pallas as pl
from jax.experimental.pallas import tpu as pltpu
from jax.experimental.pallas import tpu_sc as plsc

D_MODEL = 512
NUM_TILES = 16
TOP_K = 4
B = 4096
TERNARY_W = 0.01
SPARSITY_W = 0.005

TBLK_A = 512
TBLK_B = 512
RBLK = 512
NASSIGN = B * TOP_K
NB = (NASSIGN + NUM_TILES * (RBLK - 1)) // RBLK + 1
NPAD = NB * RBLK
NEG_INF = -3.0e38


def _dot(a, b):
    return jax.lax.dot_general(a, b, (((1,), (0,)), ((), ())),
                               preferred_element_type=jnp.float32)


def _prep_body(ints_ref, opP_ref, sa_ref, sb_ref, sc_ref, w_in_ref,
               b_in_ref, w_r_ref, b_r_ref, tri_ref, eyeT_ref,
               x_ref, topi_ref, topn_ref, lr_ref, topiT_ref, lrT_ref,
               il_ref, run_ref):
    i = pl.program_id(0)
    ints = ints_ref[...]
    op = ints[:, 0:1]
    a = ints[:, 1:2]
    b = ints[:, 2:3]
    c = ints[:, 3:4]
    T = ints.shape[0]

    @pl.when(i == 0)
    def _():
        il_ref[...] = jnp.zeros_like(il_ref)
        run_ref[...] = jnp.zeros_like(run_ref)

    ones16 = jnp.ones((NUM_TILES, 1), jnp.float32)

    oh8 = (op == lax.broadcasted_iota(jnp.int32, (1, 8), 1)
           ).astype(jnp.float32)
    bit_iota = lax.broadcasted_iota(jnp.int32, (1, 8), 1)
    a_bits = ((lax.shift_right_logical(a, bit_iota)) & 1).astype(jnp.float32)
    b_bits = ((lax.shift_right_logical(b, bit_iota)) & 1).astype(jnp.float32)
    feats = (_dot(oh8, opP_ref[...]) + _dot(a_bits, sa_ref[...]) +
             _dot(b_bits, sb_ref[...]) +
             _dot(c.astype(jnp.float32), sc_ref[...]))
    x = _dot(feats, w_in_ref[...]) + b_in_ref[...]
    x_ref[...] = x

    logits = _dot(x, w_r_ref[...]) + b_r_ref[...]
    m = jnp.max(logits, axis=1, keepdims=True)
    e = jnp.exp(logits - m)
    gates = e / _dot(e, ones16)

    iota16 = lax.broadcasted_iota(jnp.int32, (T, NUM_TILES), 1)
    v = gates
    tis, tvs = [], []
    for _ in range(TOP_K):
        mx = jnp.max(v, axis=1, keepdims=True)
        idx = jnp.min(jnp.where(v == mx, iota16, NUM_TILES), axis=1,
                      keepdims=True)
        tis.append(idx)
        tvs.append(mx)
        v = jnp.where(iota16 == idx, NEG_INF, v)
    topi = jnp.concatenate(tis, axis=1)
    topv = jnp.concatenate(tvs, axis=1)
    topn = topv / _dot(topv, jnp.ones((TOP_K, 1), jnp.float32))
    topi_ref[...] = topi
    topn_ref[...] = topn

    ohk = [(iota16 == topi[:, k:k + 1]).astype(jnp.float32)
           for k in range(TOP_K)]
    disp = ohk[0] + ohk[1] + ohk[2] + ohk[3]

    ct = _dot(tri_ref[...], disp)
    base = ct + run_ref[0:1, 0:16]
    prev = jnp.zeros((T, NUM_TILES), jnp.float32)
    lrs = []
    for k in range(TOP_K):
        lrs.append(_dot(ohk[k] * (base + prev), ones16))
        prev = prev + ohk[k]
    lr_f = jnp.concatenate(lrs, axis=1)
    lr_ref[...] = lr_f.astype(jnp.int32)
    run_ref[0:1, 0:16] += jnp.sum(disp, axis=0, keepdims=True)

    topiT_ref[...] = jax.lax.dot_general(
        topi.astype(jnp.float32), eyeT_ref[...],
        (((0,), (0,)), ((), ())),
        preferred_element_type=jnp.float32).astype(jnp.int32)
    lrT_ref[...] = jax.lax.dot_general(
        lr_f, eyeT_ref[...], (((0,), (0,)), ((), ())),
        preferred_element_type=jnp.float32).astype(jnp.int32)

    ones_col = jnp.ones((T, 1), jnp.float32)
    impcol = jax.lax.dot_general(gates, ones_col, (((0,), (0,)), ((), ())),
                                 preferred_element_type=jnp.float32)
    cntcol = jax.lax.dot_general(disp, ones_col, (((0,), (0,)), ((), ())),
                                 preferred_element_type=jnp.float32)
    il_ref[0:NUM_TILES, 0:1] += impcol
    il_ref[0:NUM_TILES, 1:2] += cntcol


def _prep_call(ints, opP, sa, sb, sc, w_in_p, b_in, w_r, b_r, tri, eyeT):
    nblk = B // TBLK_A

    def cst(s):
        return pl.BlockSpec(s, lambda i: tuple(0 for _ in s))

    def blk(s):
        return pl.BlockSpec(s, lambda i: (i,) + tuple(0 for _ in s[1:]))

    return pl.pallas_call(
        _prep_body,
        grid=(nblk,),
        in_specs=[
            blk((TBLK_A, 4)),
            cst((8, 128)),
            cst((8, 128)),
            cst((8, 128)),
            cst((1, 128)),
            cst((128, D_MODEL)),
            cst((1, D_MODEL)),
            cst((D_MODEL, NUM_TILES)),
            cst((1, NUM_TILES)),
            cst((TBLK_A, TBLK_A)),
            cst((TBLK_A, TBLK_A)),
        ],
        out_specs=[
            blk((TBLK_A, D_MODEL)),
            blk((TBLK_A, TOP_K)),
            blk((TBLK_A, TOP_K)),
            blk((TBLK_A, TOP_K)),
            pl.BlockSpec((TOP_K, TBLK_A), lambda i: (0, i)),
            pl.BlockSpec((TOP_K, TBLK_A), lambda i: (0, i)),
            cst((NUM_TILES, 128)),
        ],
        out_shape=[
            jax.ShapeDtypeStruct((B, D_MODEL), jnp.float32),
            jax.ShapeDtypeStruct((B, TOP_K), jnp.int32),
            jax.ShapeDtypeStruct((B, TOP_K), jnp.float32),
            jax.ShapeDtypeStruct((B, TOP_K), jnp.int32),
            jax.ShapeDtypeStruct((TOP_K, B), jnp.int32),
            jax.ShapeDtypeStruct((TOP_K, B), jnp.int32),
            jax.ShapeDtypeStruct((NUM_TILES, 128), jnp.float32),
        ],
        scratch_shapes=[pltpu.VMEM((8, 128), jnp.float32)],
    )(ints, opP, sa, sb, sc, w_in_p, b_in, w_r, b_r, tri, eyeT)


def _plan_body(topi_ref, lr_ref, topiT_ref, lrT_ref, il_ref,
               p_ref, pT_ref, plan_ref):
    cnt = il_ref[0:NUM_TILES, 1:2].astype(jnp.int32)
    cnt_pad = ((cnt + (RBLK - 1)) // RBLK) * RBLK
    tri16 = (lax.broadcasted_iota(jnp.int32, (NUM_TILES, NUM_TILES), 0) >
             lax.broadcasted_iota(jnp.int32, (NUM_TILES, NUM_TILES), 1)
             ).astype(jnp.float32)
    off_col = jax.lax.dot_general(
        tri16, cnt_pad.astype(jnp.float32), (((1,), (0,)), ((), ())),
        preferred_element_type=jnp.float32).astype(jnp.int32)

    eye16 = (lax.broadcasted_iota(jnp.int32, (NUM_TILES, NUM_TILES), 0) ==
             lax.broadcasted_iota(jnp.int32, (NUM_TILES, NUM_TILES), 1)
             ).astype(jnp.float32)
    off_row = jax.lax.dot_general(
        off_col.astype(jnp.float32), eye16, (((0,), (0,)), ((), ())),
        preferred_element_type=jnp.float32)
    thr = (lax.broadcasted_iota(jnp.int32, (NB, 1), 0) * RBLK
           ).astype(jnp.float32)
    cmp = (off_row <= thr).astype(jnp.float32)
    be_col = jax.lax.dot_general(
        cmp, jnp.ones((NUM_TILES, 1), jnp.float32),
        (((1,), (0,)), ((), ())),
        preferred_element_type=jnp.float32).astype(jnp.int32) - 1
    plan_ref[...] = jnp.zeros_like(plan_ref)
    plan_ref[:, 0:1] = be_col
    na = jnp.sum(cnt_pad) // RBLK
    plan_ref[0:1, 1:2] = jnp.reshape(na, (1, 1))
    plan_ref[0:NUM_TILES, 2:3] = off_col

    topi = topi_ref[...]
    offsel = jnp.zeros(topi.shape, jnp.int32)
    for e in range(NUM_TILES):
        offsel = offsel + jnp.where(topi == e, off_col[e, 0], 0)
    p_ref[...] = offsel + lr_ref[...]

    topiT = topiT_ref[...]
    offselT = jnp.zeros(topiT.shape, jnp.int32)
    for e in range(NUM_TILES):
        offselT = offselT + jnp.where(topiT == e, off_col[e, 0], 0)
    pT_ref[...] = offselT + lrT_ref[...]


def _plan_call(topi, lr, topiT, lrT, il):
    return pl.pallas_call(
        _plan_body,
        grid=(1,),
        in_specs=[
            pl.BlockSpec((B, TOP_K), lambda i: (0, 0)),
            pl.BlockSpec((B, TOP_K), lambda i: (0, 0)),
            pl.BlockSpec((TOP_K, B), lambda i: (0, 0)),
            pl.BlockSpec((TOP_K, B), lambda i: (0, 0)),
            pl.BlockSpec((NUM_TILES, 128), lambda i: (0, 0)),
        ],
        out_specs=[
            pl.BlockSpec((B, TOP_K), lambda i: (0, 0)),
            pl.BlockSpec((TOP_K, B), lambda i: (0, 0)),
            pl.BlockSpec((NB, 128), lambda i: (0, 0)),
        ],
        out_shape=[
            jax.ShapeDtypeStruct((B, TOP_K), jnp.int32),
            jax.ShapeDtypeStruct((TOP_K, B), jnp.int32),
            jax.ShapeDtypeStruct((NB, 128), jnp.int32),
        ],
    )(topi, lr, topiT, lrT, il)


SC_TOK = 64


def _make_scatter():
    info = plsc.get_sparse_core_info()
    nw = info.num_cores * info.num_subcores
    tok_per_w = B // nw
    nchunk = tok_per_w // SC_TOK
    mesh = plsc.VectorSubcoreMesh(core_axis_name="c", subcore_axis_name="s")

    @functools.partial(
        pl.kernel, mesh=mesh,
        out_type=jax.ShapeDtypeStruct((NPAD, D_MODEL), jnp.float32),
        scratch_types=[
            pltpu.VMEM((2, TOP_K, SC_TOK), jnp.int32),
            pltpu.VMEM((2, SC_TOK, D_MODEL), jnp.float32),
            pltpu.SemaphoreType.DMA,
            pltpu.SemaphoreType.DMA,
            pltpu.SemaphoreType.DMA,
        ],
    )
    def scatter_k(x_hbm, pT_hbm, xs_hbm, pidx_v, rows_v, sem_l, sem_i,
                  sem_s):
        wid = lax.axis_index("s") * info.num_cores + lax.axis_index("c")

        def load(ci, bb):
            tb = wid * tok_per_w + ci * SC_TOK
            dl = pltpu.async_copy(x_hbm.at[pl.ds(tb, SC_TOK)],
                                  rows_v.at[bb], sem_l)
            dis = [pltpu.async_copy(pT_hbm.at[k, pl.ds(tb, SC_TOK)],
                                    pidx_v.at[bb, k], sem_i)
                   for k in range(TOP_K)]
            return [dl] + dis

        pend = load(0, 0)
        scat = []
        for ci in range(nchunk):
            bb = ci & 1
            for d in pend:
                d.wait()
            if ci + 1 < nchunk:
                pend = load(ci + 1, 1 - bb)
            for d in scat:
                d.wait()
            scat = [pltpu.async_copy(rows_v.at[bb],
                                     xs_hbm.at[pidx_v.at[bb, k]], sem_s)
                    for k in range(TOP_K)]
        for d in scat:
            d.wait()

    return scatter_k


def _ffn_body(be_ref, na_ref, xs_ref, w1_ref, b1_ref, w2_ref, b2_ref,
              ys_ref):
    i = pl.program_id(0)

    @pl.when(i < na_ref[0])
    def _():
        h = jnp.maximum(
            _dot(xs_ref[...].astype(jnp.bfloat16), w1_ref[0]) + b1_ref[0],
            0.0)
        ys_ref[...] = _dot(h.astype(jnp.bfloat16), w2_ref[0]) + b2_ref[0]


def _ffn_call(be, na, xs, w1, b1, w2, b2):
    grid_spec = pltpu.PrefetchScalarGridSpec(
        num_scalar_prefetch=2,
        grid=(NB,),
        in_specs=[
            pl.BlockSpec((RBLK, D_MODEL), lambda i, be, na: (i, 0)),
            pl.BlockSpec((1, D_MODEL, D_MODEL),
                         lambda i, be, na: (be[i], 0, 0)),
            pl.BlockSpec((1, 1, D_MODEL), lambda i, be, na: (be[i], 0, 0)),
            pl.BlockSpec((1, D_MODEL, D_MODEL),
                         lambda i, be, na: (be[i], 0, 0)),
            pl.BlockSpec((1, 1, D_MODEL), lambda i, be, na: (be[i], 0, 0)),
        ],
        out_specs=pl.BlockSpec((RBLK, D_MODEL), lambda i, be, na: (i, 0)),
    )
    return pl.pallas_call(
        _ffn_body,
        grid_spec=grid_spec,
        out_shape=jax.ShapeDtypeStruct((NPAD, D_MODEL), jnp.float32),
    )(be, na, xs, w1, b1, w2, b2)


GA_CHUNK = 64


def _make_gather():
    info = plsc.get_sparse_core_info()
    nw = info.num_cores * info.num_subcores
    rows_per_w = NASSIGN // nw
    nchunk = rows_per_w // GA_CHUNK
    mesh = plsc.VectorSubcoreMesh(core_axis_name="c", subcore_axis_name="s")

    @functools.partial(
        pl.kernel, mesh=mesh,
        out_type=jax.ShapeDtypeStruct((NASSIGN, D_MODEL), jnp.float32),
        scratch_types=[
            pltpu.VMEM((2, GA_CHUNK), jnp.int32),
            pltpu.VMEM((2, GA_CHUNK, D_MODEL), jnp.float32),
            pltpu.SemaphoreType.DMA,
            pltpu.SemaphoreType.DMA,
            pltpu.SemaphoreType.DMA,
        ],
    )
    def gather_k(ys_hbm, p_hbm, y4_hbm, pidx_v, rows_v, sem_i, sem_g,
                 sem_w):
        wid = lax.axis_index("s") * info.num_cores + lax.axis_index("c")
        base = wid * rows_per_w

        def idx_load(ci, bb):
            return pltpu.async_copy(
                p_hbm.at[pl.ds(base + ci * GA_CHUNK, GA_CHUNK)],
                pidx_v.at[bb], sem_i)

        def gather_start(bb):
            return pltpu.async_copy(ys_hbm.at[pidx_v.at[bb]],
                                    rows_v.at[bb], sem_g)

        def write_start(ci, bb):
            return pltpu.async_copy(
                rows_v.at[bb],
                y4_hbm.at[pl.ds(base + ci * GA_CHUNK, GA_CHUNK)], sem_w)

        idx_load(0, 0).wait()
        gat = gather_start(0)
        idx_pend = idx_load(1, 1)
        wr = None
        for ci in range(nchunk):
            bb = ci & 1
            gat.wait()
            if ci + 1 < nchunk:
                idx_pend.wait()
                if wr is not None:
                    wr.wait()
                gat = gather_start(1 - bb)
                if ci + 2 < nchunk:
                    idx_pend = idx_load(ci + 2, bb)
            elif wr is not None:
                wr.wait()
            wr = write_start(ci, bb)
        wr.wait()

    return gather_k


def _aux_body(w1_ref, w2_ref, il_ref, out_ref):
    e = pl.program_id(0)

    @pl.when(e == 0)
    def _():
        out_ref[...] = jnp.zeros_like(out_ref)

    aw1 = jnp.abs(w1_ref[0].astype(jnp.float32))
    aw2 = jnp.abs(w2_ref[0].astype(jnp.float32))
    s = (jnp.sum(aw1 * jnp.abs(1.0 - aw1)) + jnp.sum(aw2 * jnp.abs(1.0 - aw2)))
    out_ref[0:1, 0:1] += jnp.reshape(s, (1, 1))

    @pl.when(e == NUM_TILES - 1)
    def _():
        imp = il_ref[0:NUM_TILES, 0:1] * (1.0 / B)
        load = il_ref[0:NUM_TILES, 1:2] * (1.0 / B)
        lb = NUM_TILES * jnp.sum(imp * load)
        tern = out_ref[0, 0] / (NUM_TILES * D_MODEL * D_MODEL)
        out_ref[0:1, 0:1] = jnp.reshape(
            SPARSITY_W * lb + TERNARY_W * tern, (1, 1))


def _aux_call(w1, w2, il):
    return pl.pallas_call(
        _aux_body,
        grid=(NUM_TILES,),
        in_specs=[
            pl.BlockSpec((1, D_MODEL, D_MODEL), lambda e: (e, 0, 0)),
            pl.BlockSpec((1, D_MODEL, D_MODEL), lambda e: (e, 0, 0)),
            pl.BlockSpec((NUM_TILES, 128), lambda e: (0, 0)),
        ],
        out_specs=pl.BlockSpec((8, 128), lambda e: (0, 0)),
        out_shape=jax.ShapeDtypeStruct((8, 128), jnp.float32),
    )(w1, w2, il)


def _head_body(y4_ref, topn_ref, wh1_ref, bh1_ref, wh2_ref, bh2_ref,
               rb_ref):
    T = topn_ref.shape[0]
    y3 = y4_ref[...].reshape(T, TOP_K, D_MODEL)
    topn = topn_ref[...]
    out = y3[:, 0, :] * topn[:, 0:1]
    for k in range(1, TOP_K):
        out = out + y3[:, k, :] * topn[:, k:k + 1]
    h = jnp.maximum(_dot(out, wh1_ref[...]) + bh1_ref[...], 0.0)
    z = _dot(h, wh2_ref[...]) + bh2_ref[...]
    rb_ref[...] = 1.0 / (1.0 + jnp.exp(-z))


def _head_call(y4, topn, wh1_p, bh1_p, wh2_p, bh2):
    nblk = B // TBLK_B
    return pl.pallas_call(
        _head_body,
        grid=(nblk,),
        in_specs=[
            pl.BlockSpec((TBLK_B * TOP_K, D_MODEL), lambda i: (i, 0)),
            pl.BlockSpec((TBLK_B, TOP_K), lambda i: (i, 0)),
            pl.BlockSpec((D_MODEL, 128), lambda i: (0, 0)),
            pl.BlockSpec((1, 128), lambda i: (0, 0)),
            pl.BlockSpec((128, 8), lambda i: (0, 0)),
            pl.BlockSpec((1, 8), lambda i: (0, 0)),
        ],
        out_specs=pl.BlockSpec((TBLK_B, 8), lambda i: (i, 0)),
        out_shape=jax.ShapeDtypeStruct((B, 8), jnp.float32),
    )(y4, topn, wh1_p, bh1_p, wh2_p, bh2)


def kernel(op_idx, a, b, c, op_embed, W_in, b_in, W_router, b_router,
           W1, b1, W2, b2, W_h1, b_h1, W_h2, b_h2):
    ints = jnp.stack([op_idx.astype(jnp.int32), a.astype(jnp.int32),
                      b.astype(jnp.int32), c.astype(jnp.int32)], axis=1)
    w_in_p = jnp.pad(W_in, ((0, 128 - 33), (0, 0)))
    opP = jnp.pad(op_embed, ((0, 0), (0, 128 - 16)))
    ar8 = jnp.arange(8)
    ar128 = jnp.arange(128)
    sa = (ar128[None, :] == (16 + ar8)[:, None]).astype(jnp.float32)
    sb = (ar128[None, :] == (24 + ar8)[:, None]).astype(jnp.float32)
    sc = (ar128[None, :] == 32).astype(jnp.float32).reshape(1, 128)
    wh1_p = jnp.pad(W_h1, ((0, 0), (0, 128 - 32)))
    bh1_p = jnp.pad(b_h1, (0, 128 - 32)).reshape(1, 128)
    wh2_p = jnp.pad(W_h2, ((0, 128 - 32), (0, 0)))
    tri = (jnp.arange(TBLK_A)[:, None] > jnp.arange(TBLK_A)[None, :]
           ).astype(jnp.float32)
    eyeT = jnp.eye(TBLK_A, dtype=jnp.float32)

    x_bf, topi, topn, lr, topiT, lrT, il = _prep_call(
        ints, opP, sa, sb, sc, w_in_p,
        b_in.reshape(1, D_MODEL), W_router, b_router.reshape(1, NUM_TILES),
        tri, eyeT)
    p, pT, plan = _plan_call(topi, lr, topiT, lrT, il)

    p_flat = p.reshape(NASSIGN)
    xs = _make_scatter()(x_bf, pT)
    be = plan[:, 0]
    na = plan[0:1, 1].reshape(1)
    w1_bf = W1.astype(jnp.bfloat16)
    w2_bf = W2.astype(jnp.bfloat16)
    ys = _ffn_call(be, na, xs, w1_bf,
                   b1.reshape(NUM_TILES, 1, D_MODEL),
                   w2_bf,
                   b2.reshape(NUM_TILES, 1, D_MODEL))
    y4 = _make_gather()(ys, p_flat)

    auxm = _aux_call(w1_bf, w2_bf, il)
    aux = auxm[0, 0]
    result_bits = _head_call(y4, topn, wh1_p, bh1_p, wh2_p,
                             b_h2.reshape(1, 8))
    return result_bits, topi, aux


# speedup vs baseline: 1.0586x; 1.0105x over previous
import functools

import jax
import jax.numpy as jnp
from jax import lax
from jax.experimental import pallas as pl
from jax.experimental.pallas import tpu as pltpu
from jax.experimental.pallas import tpu_sc as plsc

D_MODEL = 512
NUM_TILES = 16
TOP_K = 4
B = 4096
TERNARY_W = 0.01
SPARSITY_W = 0.005

TBLK_A = 512
TBLK_B = 512
RBLK = 512
NASSIGN = B * TOP_K
NB = (NASSIGN + NUM_TILES * (RBLK - 1)) // RBLK + 1
NPAD = NB * RBLK
NEG_INF = -3.0e38


def _dot(a, b):
    return jax.lax.dot_general(a, b, (((1,), (0,)), ((), ())),
                               preferred_element_type=jnp.float32)


def _prep_body(ints_ref, opP_ref, sa_ref, sb_ref, sc_ref, w_in_ref,
               b_in_ref, w_r_ref, b_r_ref, tri_ref, eyeT_ref,
               x_ref, topi_ref, topn_ref, lr_ref, topiT_ref, lrT_ref,
               il_ref, run_ref):
    i = pl.program_id(0)
    ints = ints_ref[...]
    op = ints[:, 0:1]
    a = ints[:, 1:2]
    b = ints[:, 2:3]
    c = ints[:, 3:4]
    T = ints.shape[0]

    @pl.when(i == 0)
    def _():
        il_ref[...] = jnp.zeros_like(il_ref)
        run_ref[...] = jnp.zeros_like(run_ref)

    ones16 = jnp.ones((NUM_TILES, 1), jnp.float32)

    oh8 = (op == lax.broadcasted_iota(jnp.int32, (1, 8), 1)
           ).astype(jnp.float32)
    bit_iota = lax.broadcasted_iota(jnp.int32, (1, 8), 1)
    a_bits = ((lax.shift_right_logical(a, bit_iota)) & 1).astype(jnp.float32)
    b_bits = ((lax.shift_right_logical(b, bit_iota)) & 1).astype(jnp.float32)
    feats = (_dot(oh8, opP_ref[...]) + _dot(a_bits, sa_ref[...]) +
             _dot(b_bits, sb_ref[...]) +
             _dot(c.astype(jnp.float32), sc_ref[...]))
    x = _dot(feats, w_in_ref[...]) + b_in_ref[...]
    x_ref[...] = x

    logits = _dot(x, w_r_ref[...]) + b_r_ref[...]
    m = jnp.max(logits, axis=1, keepdims=True)
    e = jnp.exp(logits - m)
    gates = e / _dot(e, ones16)

    iota16 = lax.broadcasted_iota(jnp.int32, (T, NUM_TILES), 1)
    v = gates
    tis, tvs = [], []
    for _ in range(TOP_K):
        mx = jnp.max(v, axis=1, keepdims=True)
        idx = jnp.min(jnp.where(v == mx, iota16, NUM_TILES), axis=1,
                      keepdims=True)
        tis.append(idx)
        tvs.append(mx)
        v = jnp.where(iota16 == idx, NEG_INF, v)
    topi = jnp.concatenate(tis, axis=1)
    topv = jnp.concatenate(tvs, axis=1)
    topn = topv / _dot(topv, jnp.ones((TOP_K, 1), jnp.float32))
    topi_ref[...] = topi
    topn_ref[...] = topn

    ohk = [(iota16 == topi[:, k:k + 1]).astype(jnp.float32)
           for k in range(TOP_K)]
    disp = ohk[0] + ohk[1] + ohk[2] + ohk[3]

    ct = _dot(tri_ref[...], disp)
    base = ct + run_ref[0:1, 0:16]
    prev = jnp.zeros((T, NUM_TILES), jnp.float32)
    lrs = []
    for k in range(TOP_K):
        lrs.append(_dot(ohk[k] * (base + prev), ones16))
        prev = prev + ohk[k]
    lr_f = jnp.concatenate(lrs, axis=1)
    lr_ref[...] = lr_f.astype(jnp.int32)
    run_ref[0:1, 0:16] += jnp.sum(disp, axis=0, keepdims=True)

    topiT_ref[...] = jax.lax.dot_general(
        topi.astype(jnp.float32), eyeT_ref[...],
        (((0,), (0,)), ((), ())),
        preferred_element_type=jnp.float32).astype(jnp.int32)
    lrT_ref[...] = jax.lax.dot_general(
        lr_f, eyeT_ref[...], (((0,), (0,)), ((), ())),
        preferred_element_type=jnp.float32).astype(jnp.int32)

    ones_col = jnp.ones((T, 1), jnp.float32)
    impcol = jax.lax.dot_general(gates, ones_col, (((0,), (0,)), ((), ())),
                                 preferred_element_type=jnp.float32)
    cntcol = jax.lax.dot_general(disp, ones_col, (((0,), (0,)), ((), ())),
                                 preferred_element_type=jnp.float32)
    il_ref[0:NUM_TILES, 0:1] += impcol
    il_ref[0:NUM_TILES, 1:2] += cntcol


def _prep_call(ints, opP, sa, sb, sc, w_in_p, b_in, w_r, b_r, tri, eyeT):
    nblk = B // TBLK_A

    def cst(s):
        return pl.BlockSpec(s, lambda i: tuple(0 for _ in s))

    def blk(s):
        return pl.BlockSpec(s, lambda i: (i,) + tuple(0 for _ in s[1:]))

    return pl.pallas_call(
        _prep_body,
        grid=(nblk,),
        in_specs=[
            blk((TBLK_A, 4)),
            cst((8, 128)),
            cst((8, 128)),
            cst((8, 128)),
            cst((1, 128)),
            cst((128, D_MODEL)),
            cst((1, D_MODEL)),
            cst((D_MODEL, NUM_TILES)),
            cst((1, NUM_TILES)),
            cst((TBLK_A, TBLK_A)),
            cst((TBLK_A, TBLK_A)),
        ],
        out_specs=[
            blk((TBLK_A, D_MODEL)),
            blk((TBLK_A, TOP_K)),
            blk((TBLK_A, TOP_K)),
            blk((TBLK_A, TOP_K)),
            pl.BlockSpec((TOP_K, TBLK_A), lambda i: (0, i)),
            pl.BlockSpec((TOP_K, TBLK_A), lambda i: (0, i)),
            cst((NUM_TILES, 128)),
        ],
        out_shape=[
            jax.ShapeDtypeStruct((B, D_MODEL), jnp.float32),
            jax.ShapeDtypeStruct((B, TOP_K), jnp.int32),
            jax.ShapeDtypeStruct((B, TOP_K), jnp.float32),
            jax.ShapeDtypeStruct((B, TOP_K), jnp.int32),
            jax.ShapeDtypeStruct((TOP_K, B), jnp.int32),
            jax.ShapeDtypeStruct((TOP_K, B), jnp.int32),
            jax.ShapeDtypeStruct((NUM_TILES, 128), jnp.float32),
        ],
        scratch_shapes=[pltpu.VMEM((8, 128), jnp.float32)],
    )(ints, opP, sa, sb, sc, w_in_p, b_in, w_r, b_r, tri, eyeT)


def _plan_body(topi_ref, lr_ref, topiT_ref, lrT_ref, il_ref,
               p_ref, pT_ref, plan_ref):
    cnt = il_ref[0:NUM_TILES, 1:2].astype(jnp.int32)
    cnt_pad = ((cnt + (RBLK - 1)) // RBLK) * RBLK
    tri16 = (lax.broadcasted_iota(jnp.int32, (NUM_TILES, NUM_TILES), 0) >
             lax.broadcasted_iota(jnp.int32, (NUM_TILES, NUM_TILES), 1)
             ).astype(jnp.float32)
    off_col = jax.lax.dot_general(
        tri16, cnt_pad.astype(jnp.float32), (((1,), (0,)), ((), ())),
        preferred_element_type=jnp.float32).astype(jnp.int32)

    eye16 = (lax.broadcasted_iota(jnp.int32, (NUM_TILES, NUM_TILES), 0) ==
             lax.broadcasted_iota(jnp.int32, (NUM_TILES, NUM_TILES), 1)
             ).astype(jnp.float32)
    off_row = jax.lax.dot_general(
        off_col.astype(jnp.float32), eye16, (((0,), (0,)), ((), ())),
        preferred_element_type=jnp.float32)
    thr = (lax.broadcasted_iota(jnp.int32, (NB, 1), 0) * RBLK
           ).astype(jnp.float32)
    cmp = (off_row <= thr).astype(jnp.float32)
    be_col = jax.lax.dot_general(
        cmp, jnp.ones((NUM_TILES, 1), jnp.float32),
        (((1,), (0,)), ((), ())),
        preferred_element_type=jnp.float32).astype(jnp.int32) - 1
    plan_ref[...] = jnp.zeros_like(plan_ref)
    plan_ref[:, 0:1] = be_col
    na = jnp.sum(cnt_pad) // RBLK
    plan_ref[0:1, 1:2] = jnp.reshape(na, (1, 1))
    plan_ref[0:NUM_TILES, 2:3] = off_col

    topi = topi_ref[...]
    offsel = jnp.zeros(topi.shape, jnp.int32)
    for e in range(NUM_TILES):
        offsel = offsel + jnp.where(topi == e, off_col[e, 0], 0)
    p_ref[...] = offsel + lr_ref[...]

    topiT = topiT_ref[...]
    offselT = jnp.zeros(topiT.shape, jnp.int32)
    for e in range(NUM_TILES):
        offselT = offselT + jnp.where(topiT == e, off_col[e, 0], 0)
    pT_ref[...] = offselT + lrT_ref[...]


def _plan_call(topi, lr, topiT, lrT, il):
    return pl.pallas_call(
        _plan_body,
        grid=(1,),
        in_specs=[
            pl.BlockSpec((B, TOP_K), lambda i: (0, 0)),
            pl.BlockSpec((B, TOP_K), lambda i: (0, 0)),
            pl.BlockSpec((TOP_K, B), lambda i: (0, 0)),
            pl.BlockSpec((TOP_K, B), lambda i: (0, 0)),
            pl.BlockSpec((NUM_TILES, 128), lambda i: (0, 0)),
        ],
        out_specs=[
            pl.BlockSpec((B, TOP_K), lambda i: (0, 0)),
            pl.BlockSpec((TOP_K, B), lambda i: (0, 0)),
            pl.BlockSpec((NB, 128), lambda i: (0, 0)),
        ],
        out_shape=[
            jax.ShapeDtypeStruct((B, TOP_K), jnp.int32),
            jax.ShapeDtypeStruct((TOP_K, B), jnp.int32),
            jax.ShapeDtypeStruct((NB, 128), jnp.int32),
        ],
    )(topi, lr, topiT, lrT, il)


SC_TOK = 64


def _make_scatter():
    info = plsc.get_sparse_core_info()
    nw = info.num_cores * info.num_subcores
    tok_per_w = B // nw
    nchunk = tok_per_w // SC_TOK
    mesh = plsc.VectorSubcoreMesh(core_axis_name="c", subcore_axis_name="s")

    @functools.partial(
        pl.kernel, mesh=mesh,
        out_type=jax.ShapeDtypeStruct((NPAD, D_MODEL), jnp.float32),
        scratch_types=[
            pltpu.VMEM((2, TOP_K, SC_TOK), jnp.int32),
            pltpu.VMEM((2, SC_TOK, D_MODEL), jnp.float32),
            pltpu.SemaphoreType.DMA,
            pltpu.SemaphoreType.DMA,
            pltpu.SemaphoreType.DMA,
        ],
    )
    def scatter_k(x_hbm, pT_hbm, xs_hbm, pidx_v, rows_v, sem_l, sem_i,
                  sem_s):
        wid = lax.axis_index("s") * info.num_cores + lax.axis_index("c")

        def load(ci, bb):
            tb = wid * tok_per_w + ci * SC_TOK
            dl = pltpu.async_copy(x_hbm.at[pl.ds(tb, SC_TOK)],
                                  rows_v.at[bb], sem_l)
            dis = [pltpu.async_copy(pT_hbm.at[k, pl.ds(tb, SC_TOK)],
                                    pidx_v.at[bb, k], sem_i)
                   for k in range(TOP_K)]
            return [dl] + dis

        pend = load(0, 0)
        scat = []
        for ci in range(nchunk):
            bb = ci & 1
            for d in pend:
                d.wait()
            if ci + 1 < nchunk:
                pend = load(ci + 1, 1 - bb)
            for d in scat:
                d.wait()
            scat = [pltpu.async_copy(rows_v.at[bb],
                                     xs_hbm.at[pidx_v.at[bb, k]], sem_s)
                    for k in range(TOP_K)]
        for d in scat:
            d.wait()

    return scatter_k


def _ffn_body(be_ref, na_ref, xs_ref, w1_ref, b1_ref, w2_ref, b2_ref,
              ys_ref):
    i = pl.program_id(0)

    @pl.when(i < na_ref[0])
    def _():
        h = jnp.maximum(
            _dot(xs_ref[...].astype(jnp.bfloat16), w1_ref[0]) + b1_ref[0],
            0.0)
        ys_ref[...] = _dot(h.astype(jnp.bfloat16), w2_ref[0]) + b2_ref[0]


def _ffn_call(be, na, xs, w1, b1, w2, b2):
    grid_spec = pltpu.PrefetchScalarGridSpec(
        num_scalar_prefetch=2,
        grid=(NB,),
        in_specs=[
            pl.BlockSpec((RBLK, D_MODEL),
                         lambda i, be, na: (jnp.minimum(i, na[0] - 1), 0)),
            pl.BlockSpec((1, D_MODEL, D_MODEL),
                         lambda i, be, na: (be[i], 0, 0)),
            pl.BlockSpec((1, 1, D_MODEL), lambda i, be, na: (be[i], 0, 0)),
            pl.BlockSpec((1, D_MODEL, D_MODEL),
                         lambda i, be, na: (be[i], 0, 0)),
            pl.BlockSpec((1, 1, D_MODEL), lambda i, be, na: (be[i], 0, 0)),
        ],
        out_specs=pl.BlockSpec((RBLK, D_MODEL), lambda i, be, na: (i, 0)),
    )
    return pl.pallas_call(
        _ffn_body,
        grid_spec=grid_spec,
        out_shape=jax.ShapeDtypeStruct((NPAD, D_MODEL), jnp.float32),
    )(be, na, xs, w1, b1, w2, b2)


GA_CHUNK = 64


def _make_gather():
    info = plsc.get_sparse_core_info()
    nw = info.num_cores * info.num_subcores
    rows_per_w = NASSIGN // nw
    nchunk = rows_per_w // GA_CHUNK
    mesh = plsc.VectorSubcoreMesh(core_axis_name="c", subcore_axis_name="s")

    @functools.partial(
        pl.kernel, mesh=mesh,
        out_type=jax.ShapeDtypeStruct((NASSIGN, D_MODEL), jnp.float32),
        scratch_types=[
            pltpu.VMEM((2, GA_CHUNK), jnp.int32),
            pltpu.VMEM((2, GA_CHUNK, D_MODEL), jnp.float32),
            pltpu.SemaphoreType.DMA,
            pltpu.SemaphoreType.DMA,
            pltpu.SemaphoreType.DMA,
        ],
    )
    def gather_k(ys_hbm, p_hbm, y4_hbm, pidx_v, rows_v, sem_i, sem_g,
                 sem_w):
        wid = lax.axis_index("s") * info.num_cores + lax.axis_index("c")
        base = wid * rows_per_w

        def idx_load(ci, bb):
            return pltpu.async_copy(
                p_hbm.at[pl.ds(base + ci * GA_CHUNK, GA_CHUNK)],
                pidx_v.at[bb], sem_i)

        def gather_start(bb):
            return pltpu.async_copy(ys_hbm.at[pidx_v.at[bb]],
                                    rows_v.at[bb], sem_g)

        def write_start(ci, bb):
            return pltpu.async_copy(
                rows_v.at[bb],
                y4_hbm.at[pl.ds(base + ci * GA_CHUNK, GA_CHUNK)], sem_w)

        idx_load(0, 0).wait()
        gat = gather_start(0)
        idx_pend = idx_load(1, 1)
        wr = None
        for ci in range(nchunk):
            bb = ci & 1
            gat.wait()
            if ci + 1 < nchunk:
                idx_pend.wait()
                if wr is not None:
                    wr.wait()
                gat = gather_start(1 - bb)
                if ci + 2 < nchunk:
                    idx_pend = idx_load(ci + 2, bb)
            elif wr is not None:
                wr.wait()
            wr = write_start(ci, bb)
        wr.wait()

    return gather_k


def _aux_body(w1_ref, w2_ref, il_ref, out_ref):
    e = pl.program_id(0)

    @pl.when(e == 0)
    def _():
        out_ref[...] = jnp.zeros_like(out_ref)

    aw1 = jnp.abs(w1_ref[0].astype(jnp.float32))
    aw2 = jnp.abs(w2_ref[0].astype(jnp.float32))
    s = (jnp.sum(aw1 * jnp.abs(1.0 - aw1)) + jnp.sum(aw2 * jnp.abs(1.0 - aw2)))
    out_ref[0:1, 0:1] += jnp.reshape(s, (1, 1))

    @pl.when(e == NUM_TILES - 1)
    def _():
        imp = il_ref[0:NUM_TILES, 0:1] * (1.0 / B)
        load = il_ref[0:NUM_TILES, 1:2] * (1.0 / B)
        lb = NUM_TILES * jnp.sum(imp * load)
        tern = out_ref[0, 0] / (NUM_TILES * D_MODEL * D_MODEL)
        out_ref[0:1, 0:1] = jnp.reshape(
            SPARSITY_W * lb + TERNARY_W * tern, (1, 1))


def _aux_call(w1, w2, il):
    return pl.pallas_call(
        _aux_body,
        grid=(NUM_TILES,),
        in_specs=[
            pl.BlockSpec((1, D_MODEL, D_MODEL), lambda e: (e, 0, 0)),
            pl.BlockSpec((1, D_MODEL, D_MODEL), lambda e: (e, 0, 0)),
            pl.BlockSpec((NUM_TILES, 128), lambda e: (0, 0)),
        ],
        out_specs=pl.BlockSpec((8, 128), lambda e: (0, 0)),
        out_shape=jax.ShapeDtypeStruct((8, 128), jnp.float32),
    )(w1, w2, il)


def _head_body(y4_ref, topn_ref, wh1_ref, bh1_ref, wh2_ref, bh2_ref,
               rb_ref):
    T = topn_ref.shape[0]
    y3 = y4_ref[...].reshape(T, TOP_K, D_MODEL)
    topn = topn_ref[...]
    out = y3[:, 0, :] * topn[:, 0:1]
    for k in range(1, TOP_K):
        out = out + y3[:, k, :] * topn[:, k:k + 1]
    h = jnp.maximum(_dot(out, wh1_ref[...]) + bh1_ref[...], 0.0)
    z = _dot(h, wh2_ref[...]) + bh2_ref[...]
    rb_ref[...] = 1.0 / (1.0 + jnp.exp(-z))


def _head_call(y4, topn, wh1_p, bh1_p, wh2_p, bh2):
    nblk = B // TBLK_B
    return pl.pallas_call(
        _head_body,
        grid=(nblk,),
        in_specs=[
            pl.BlockSpec((TBLK_B * TOP_K, D_MODEL), lambda i: (i, 0)),
            pl.BlockSpec((TBLK_B, TOP_K), lambda i: (i, 0)),
            pl.BlockSpec((D_MODEL, 128), lambda i: (0, 0)),
            pl.BlockSpec((1, 128), lambda i: (0, 0)),
            pl.BlockSpec((128, 8), lambda i: (0, 0)),
            pl.BlockSpec((1, 8), lambda i: (0, 0)),
        ],
        out_specs=pl.BlockSpec((TBLK_B, 8), lambda i: (i, 0)),
        out_shape=jax.ShapeDtypeStruct((B, 8), jnp.float32),
    )(y4, topn, wh1_p, bh1_p, wh2_p, bh2)


def kernel(op_idx, a, b, c, op_embed, W_in, b_in, W_router, b_router,
           W1, b1, W2, b2, W_h1, b_h1, W_h2, b_h2):
    ints = jnp.stack([op_idx.astype(jnp.int32), a.astype(jnp.int32),
                      b.astype(jnp.int32), c.astype(jnp.int32)], axis=1)
    w_in_p = jnp.pad(W_in, ((0, 128 - 33), (0, 0)))
    opP = jnp.pad(op_embed, ((0, 0), (0, 128 - 16)))
    ar8 = jnp.arange(8)
    ar128 = jnp.arange(128)
    sa = (ar128[None, :] == (16 + ar8)[:, None]).astype(jnp.float32)
    sb = (ar128[None, :] == (24 + ar8)[:, None]).astype(jnp.float32)
    sc = (ar128[None, :] == 32).astype(jnp.float32).reshape(1, 128)
    wh1_p = jnp.pad(W_h1, ((0, 0), (0, 128 - 32)))
    bh1_p = jnp.pad(b_h1, (0, 128 - 32)).reshape(1, 128)
    wh2_p = jnp.pad(W_h2, ((0, 128 - 32), (0, 0)))
    tri = (jnp.arange(TBLK_A)[:, None] > jnp.arange(TBLK_A)[None, :]
           ).astype(jnp.float32)
    eyeT = jnp.eye(TBLK_A, dtype=jnp.float32)

    x_bf, topi, topn, lr, topiT, lrT, il = _prep_call(
        ints, opP, sa, sb, sc, w_in_p,
        b_in.reshape(1, D_MODEL), W_router, b_router.reshape(1, NUM_TILES),
        tri, eyeT)
    p, pT, plan = _plan_call(topi, lr, topiT, lrT, il)

    p_flat = p.reshape(NASSIGN)
    xs = _make_scatter()(x_bf, pT)
    be = plan[:, 0]
    na = plan[0:1, 1].reshape(1)
    w1_bf = W1.astype(jnp.bfloat16)
    w2_bf = W2.astype(jnp.bfloat16)
    ys = _ffn_call(be, na, xs, w1_bf,
                   b1.reshape(NUM_TILES, 1, D_MODEL),
                   w2_bf,
                   b2.reshape(NUM_TILES, 1, D_MODEL))
    auxm = _aux_call(w1_bf, w2_bf, il)
    y4 = _make_gather()(ys, p_flat)
    aux = auxm[0, 0]
    result_bits = _head_call(y4, topn, wh1_p, bh1_p, wh2_p,
                             b_h2.reshape(1, 8))
    return result_bits, topi, aux
